# baseline (speedup 1.0000x reference)
import functools

import jax
import jax.numpy as jnp
from jax import lax
from jax.experimental import pallas as pl
from jax.experimental.pallas import tpu as pltpu
from jax.experimental.pallas import tpu_sc as plsc

N = 10000
E = 320000
D_IN = 128
D_HID = 16
D_OUT = 7

N_PAD = 10240
NW = 32
EPW = E // NW
NCHUNK = 80
CW = EPW // NCHUNK
ROWS = N_PAD // 16

_mesh = plsc.VectorSubcoreMesh(core_axis_name="c", subcore_axis_name="s")


@functools.partial(
    pl.kernel,
    out_type=jax.ShapeDtypeStruct((NW, N_PAD), jnp.float32),
    mesh=_mesh,
    scratch_types=[
        pltpu.VMEM((EPW,), jnp.int32),
        pltpu.VMEM((N_PAD,), jnp.float32),
    ],
    compiler_params=pltpu.CompilerParams(needs_layout_passes=False),
)
def _deg_kernel(e3_hbm, out_hbm, dvm, deg_l):
    cid = lax.axis_index("c")
    sid = lax.axis_index("s")
    wid = cid * 16 + sid
    pltpu.sync_copy(e3_hbm.at[1, wid], dvm)

    zero16 = jnp.zeros((16,), jnp.float32)
    ones16 = jnp.ones((16,), jnp.float32)

    def _zero_deg(i, c):
        deg_l[pl.ds(i * 16, 16)] = zero16
        return c
    lax.fori_loop(0, ROWS, _zero_deg, 0)

    def _hist16(g, c):
        idx = dvm[pl.ds(g * 16, 16)]
        plsc.addupdate_scatter(deg_l, [idx], ones16)
        return c
    lax.fori_loop(0, EPW // 16, _hist16, 0)

    pltpu.sync_copy(deg_l, out_hbm.at[wid])


@functools.partial(
    pl.kernel,
    out_type=jax.ShapeDtypeStruct((2, N_PAD, 16), jnp.float32),
    mesh=_mesh,
    scratch_types=[
        pltpu.VMEM((NCHUNK, CW), jnp.int32),
        pltpu.VMEM((NCHUNK, CW), jnp.int32),
        pltpu.VMEM((8 * CW, 16), jnp.float32),
        pltpu.VMEM((N_PAD // 16, 16), jnp.float32),
        pltpu.VMEM_SHARED((N_PAD, 16), jnp.float32),
        pltpu.SemaphoreType.DMA((8,)),
        pltpu.SemaphoreType.DMA((8,)),
    ],
    compiler_params=pltpu.CompilerParams(use_tc_tiling_on_sc=False),
)
def _agg_kernel(y_hbm, e4_hbm, out_hbm, svm, dvm, gb, obuf, agg, gsem, ssem):
    cid = lax.axis_index("c")
    sid = lax.axis_index("s")
    wid = cid * 16 + sid
    slab = N_PAD // 16
    pltpu.sync_copy(e4_hbm.at[0, wid], svm)
    pltpu.sync_copy(e4_hbm.at[1, wid], dvm)

    zero16 = jnp.zeros((16,), jnp.float32)

    def _zero_obuf(i, c):
        obuf[i, :] = zero16
        return c
    lax.fori_loop(0, slab, _zero_obuf, 0)
    pltpu.sync_copy(obuf, agg.at[pl.ds(sid * slab, slab)])
    plsc.subcore_barrier()

    def _group(i, c):
        base = i * 8
        for k in range(8):
            j = base + k
            buf_k = gb.at[pl.ds(k * CW, CW)]
            km = (k - 2) % 8
            buf_m = gb.at[pl.ds(km * CW, CW)]

            @pl.when(j >= 8)
            def _():
                pltpu.make_async_copy(
                    buf_k, agg.at[dvm.at[j - 8]], ssem.at[k]).wait()

            @pl.when(j < NCHUNK)
            def _():
                pltpu.async_copy(y_hbm.at[svm.at[j]], buf_k, gsem.at[k])

            @pl.when(jnp.logical_and(j >= 2, j < NCHUNK + 2))
            def _():
                pltpu.make_async_copy(
                    y_hbm.at[svm.at[j - 2]], buf_m, gsem.at[km]).wait()
                pltpu.async_copy(
                    buf_m, agg.at[dvm.at[j - 2]], ssem.at[km], add=True)
        return c
    lax.fori_loop(0, (NCHUNK + 8) // 8, _group, 0)

    plsc.subcore_barrier()
    pltpu.sync_copy(agg.at[pl.ds(sid * slab, slab)], obuf)
    pltpu.sync_copy(obuf, out_hbm.at[cid, pl.ds(sid * slab, slab)])


def _tc0_body(x_ref, w_ref, xw_ref):
    xw_ref[...] = jnp.dot(x_ref[...], w_ref[...],
                          preferred_element_type=jnp.float32)


_tc0 = pl.pallas_call(
    _tc0_body,
    out_shape=jax.ShapeDtypeStruct((N, 16), jnp.float32),
)


def _tc1_body(xw_ref, dall_ref, y_ref, dinv_ref):
    deg = jnp.sum(dall_ref[...], axis=0)[:N, None] + 1.0
    dv = lax.rsqrt(deg)
    y_ref[...] = dv * xw_ref[...]
    dinv_ref[...] = dv


_tc1 = pl.pallas_call(
    _tc1_body,
    out_shape=[
        jax.ShapeDtypeStruct((N, 16), jnp.float32),
        jax.ShapeDtypeStruct((N, 1), jnp.float32),
    ],
)


def _tc2_body(a_ref, y1_ref, dinv_ref, b1_ref, w2_ref, y2_ref):
    a = a_ref[...]
    dv = dinv_ref[...]
    pre = (a[0, :N] + a[1, :N] + y1_ref[...]) * dv + b1_ref[...]
    h = jnp.maximum(pre, 0.0)
    hw = jnp.dot(h, w2_ref[...], preferred_element_type=jnp.float32)
    y2_ref[...] = dv * hw


_tc2 = pl.pallas_call(
    _tc2_body,
    out_shape=jax.ShapeDtypeStruct((N, 16), jnp.float32),
)


def _tc3_body(a_ref, y2_ref, dinv_ref, b2_ref, o_ref):
    a = a_ref[...]
    t = (a[0, :N] + a[1, :N] + y2_ref[...]) * dinv_ref[...] + b2_ref[...]
    col = lax.broadcasted_iota(jnp.int32, (N, 16), 1)
    valid = col < D_OUT
    tm = jnp.where(valid, t, jnp.float32(-1e30))
    mx = jnp.max(tm, axis=1, keepdims=True)
    e = jnp.where(valid, jnp.exp(t - mx), 0.0)
    s = jnp.sum(e, axis=1, keepdims=True)
    o_ref[...] = t - mx - jnp.log(s)


_tc3 = pl.pallas_call(
    _tc3_body,
    out_shape=jax.ShapeDtypeStruct((N, 16), jnp.float32),
)


def kernel(x, edge_index, W1, b1, W2, b2):
    e3 = edge_index.reshape(2, NW, EPW)
    e4 = edge_index.reshape(2, NW, NCHUNK, CW)

    w2p = jnp.pad(W2, ((0, 0), (0, 16 - D_OUT)))
    b1p = b1.reshape(1, 16)
    b2p = jnp.pad(b2, (0, 16 - D_OUT)).reshape(1, 16)

    d_all = _deg_kernel(e3)
    xw = _tc0(x, W1)
    y1, dinv = _tc1(xw, d_all)
    a = _agg_kernel(y1, e4)
    y2 = _tc2(a, y1, dinv, b1p, w2p)
    a2 = _agg_kernel(y2, e4)
    outf = _tc3(a2, y2, dinv, b2p)
    return outf[:, :D_OUT]


# speedup vs baseline: 1.1451x; 1.1451x over previous
import functools

import jax
import jax.numpy as jnp
from jax import lax
from jax.experimental import pallas as pl
from jax.experimental.pallas import tpu as pltpu
from jax.experimental.pallas import tpu_sc as plsc

N = 10000
E = 320000
D_IN = 128
D_HID = 16
D_OUT = 7

N_PAD = 10240
NW = 32
EPW = E // NW
NCHUNK = 80
CW = EPW // NCHUNK
ROWS = N_PAD // 16

_mesh = plsc.VectorSubcoreMesh(core_axis_name="c", subcore_axis_name="s")


@functools.partial(
    pl.kernel,
    out_type=jax.ShapeDtypeStruct((NW, N_PAD), jnp.float32),
    mesh=_mesh,
    scratch_types=[
        pltpu.VMEM((EPW,), jnp.int32),
        pltpu.VMEM((N_PAD,), jnp.float32),
    ],
    compiler_params=pltpu.CompilerParams(needs_layout_passes=False),
)
def _deg_kernel(e3_hbm, out_hbm, dvm, deg_l):
    cid = lax.axis_index("c")
    sid = lax.axis_index("s")
    wid = cid * 16 + sid
    pltpu.sync_copy(e3_hbm.at[1, wid], dvm)

    zero16 = jnp.zeros((16,), jnp.float32)
    ones16 = jnp.ones((16,), jnp.float32)

    def _zero_deg(i, c):
        deg_l[pl.ds(i * 16, 16)] = zero16
        return c
    lax.fori_loop(0, ROWS, _zero_deg, 0)

    def _hist16(g, c):
        idx = dvm[pl.ds(g * 16, 16)]
        plsc.addupdate_scatter(deg_l, [idx], ones16)
        return c
    lax.fori_loop(0, EPW // 16, _hist16, 0)

    pltpu.sync_copy(deg_l, out_hbm.at[wid])


@functools.partial(
    pl.kernel,
    out_type=jax.ShapeDtypeStruct((2, N_PAD, 16), jnp.float32),
    mesh=_mesh,
    scratch_types=[
        pltpu.VMEM((NCHUNK, CW), jnp.int32),
        pltpu.VMEM((NCHUNK, CW), jnp.int32),
        pltpu.VMEM((8 * CW, 16), jnp.float32),
        pltpu.VMEM((N_PAD // 16, 16), jnp.float32),
        pltpu.VMEM_SHARED((N_PAD, 16), jnp.float32),
        pltpu.SemaphoreType.DMA((8,)),
        pltpu.SemaphoreType.DMA((8,)),
    ],
    compiler_params=pltpu.CompilerParams(use_tc_tiling_on_sc=False),
)
def _agg_kernel(y_hbm, e4_hbm, out_hbm, svm, dvm, gb, obuf, agg, gsem, ssem):
    cid = lax.axis_index("c")
    sid = lax.axis_index("s")
    wid = cid * 16 + sid
    slab = N_PAD // 16
    pltpu.sync_copy(e4_hbm.at[0, wid], svm)
    pltpu.sync_copy(e4_hbm.at[1, wid], dvm)

    zero16 = jnp.zeros((16,), jnp.float32)

    def _zero_obuf(i, c):
        obuf[i, :] = zero16
        return c
    lax.fori_loop(0, slab, _zero_obuf, 0)
    pltpu.sync_copy(obuf, agg.at[pl.ds(sid * slab, slab)])
    plsc.subcore_barrier()

    def _group(i, c):
        base = i * 8
        for k in range(8):
            j = base + k
            buf_k = gb.at[pl.ds(k * CW, CW)]
            km = (k - 4) % 8
            buf_m = gb.at[pl.ds(km * CW, CW)]

            @pl.when(j >= 8)
            def _():
                pltpu.make_async_copy(
                    buf_k, agg.at[dvm.at[j - 8]], ssem.at[k]).wait()

            @pl.when(j < NCHUNK)
            def _():
                pltpu.async_copy(y_hbm.at[svm.at[j]], buf_k, gsem.at[k])

            @pl.when(jnp.logical_and(j >= 4, j < NCHUNK + 4))
            def _():
                pltpu.make_async_copy(
                    y_hbm.at[svm.at[j - 4]], buf_m, gsem.at[km]).wait()
                pltpu.async_copy(
                    buf_m, agg.at[dvm.at[j - 4]], ssem.at[km], add=True)
        return c
    lax.fori_loop(0, (NCHUNK + 8) // 8, _group, 0)

    plsc.subcore_barrier()
    pltpu.sync_copy(agg.at[pl.ds(sid * slab, slab)], obuf)
    pltpu.sync_copy(obuf, out_hbm.at[cid, pl.ds(sid * slab, slab)])


def _tc1_body(x_ref, w_ref, dall_ref, y_ref, dinv_ref):
    deg = jnp.sum(dall_ref[...], axis=0)[:N, None] + 1.0
    dv = lax.rsqrt(deg)
    xw = jnp.dot(x_ref[...], w_ref[...], preferred_element_type=jnp.float32)
    y_ref[...] = dv * xw
    dinv_ref[...] = dv


_tc1 = pl.pallas_call(
    _tc1_body,
    out_shape=[
        jax.ShapeDtypeStruct((N, 16), jnp.float32),
        jax.ShapeDtypeStruct((N, 1), jnp.float32),
    ],
)


def _tc2_body(a_ref, y1_ref, dinv_ref, b1_ref, w2_ref, y2_ref):
    a = a_ref[...]
    dv = dinv_ref[...]
    pre = (a[0, :N] + a[1, :N] + y1_ref[...]) * dv + b1_ref[...]
    h = jnp.maximum(pre, 0.0)
    hw = jnp.dot(h, w2_ref[...], preferred_element_type=jnp.float32)
    y2_ref[...] = dv * hw


_tc2 = pl.pallas_call(
    _tc2_body,
    out_shape=jax.ShapeDtypeStruct((N, 16), jnp.float32),
)


def _tc3_body(a_ref, y2_ref, dinv_ref, b2_ref, o_ref):
    a = a_ref[...]
    t = (a[0, :N] + a[1, :N] + y2_ref[...]) * dinv_ref[...] + b2_ref[...]
    col = lax.broadcasted_iota(jnp.int32, (N, 16), 1)
    valid = col < D_OUT
    tm = jnp.where(valid, t, jnp.float32(-1e30))
    mx = jnp.max(tm, axis=1, keepdims=True)
    e = jnp.where(valid, jnp.exp(t - mx), 0.0)
    s = jnp.sum(e, axis=1, keepdims=True)
    o_ref[...] = t - mx - jnp.log(s)


_tc3 = pl.pallas_call(
    _tc3_body,
    out_shape=jax.ShapeDtypeStruct((N, 16), jnp.float32),
)


def kernel(x, edge_index, W1, b1, W2, b2):
    e3 = edge_index.reshape(2, NW, EPW)
    e4 = edge_index.reshape(2, NW, NCHUNK, CW)

    w2p = jnp.pad(W2, ((0, 0), (0, 16 - D_OUT)))
    b1p = b1.reshape(1, 16)
    b2p = jnp.pad(b2, (0, 16 - D_OUT)).reshape(1, 16)

    d_all = _deg_kernel(e3)
    y1, dinv = _tc1(x, W1, d_all)
    a = _agg_kernel(y1, e4)
    y2 = _tc2(a, y1, dinv, b1p, w2p)
    a2 = _agg_kernel(y2, e4)
    outf = _tc3(a2, y2, dinv, b2p)
    return outf[:, :D_OUT]


# speedup vs baseline: 1.1460x; 1.0008x over previous
import functools

import jax
import jax.numpy as jnp
from jax import lax
from jax.experimental import pallas as pl
from jax.experimental.pallas import tpu as pltpu
from jax.experimental.pallas import tpu_sc as plsc

N = 10000
E = 320000
D_IN = 128
D_HID = 16
D_OUT = 7

N_PAD = 10240
NW = 32
EPW = E // NW
NCHUNK = 80
CW = EPW // NCHUNK
ROWS = N_PAD // 16

_mesh = plsc.VectorSubcoreMesh(core_axis_name="c", subcore_axis_name="s")


@functools.partial(
    pl.kernel,
    out_type=jax.ShapeDtypeStruct((NW, N_PAD), jnp.float32),
    mesh=_mesh,
    scratch_types=[
        pltpu.VMEM((EPW,), jnp.int32),
        pltpu.VMEM((N_PAD,), jnp.float32),
    ],
    compiler_params=pltpu.CompilerParams(needs_layout_passes=False),
)
def _deg_kernel(e3_hbm, out_hbm, dvm, deg_l):
    cid = lax.axis_index("c")
    sid = lax.axis_index("s")
    wid = cid * 16 + sid
    pltpu.sync_copy(e3_hbm.at[1, wid], dvm)

    zero16 = jnp.zeros((16,), jnp.float32)
    ones16 = jnp.ones((16,), jnp.float32)

    def _zero_deg(i, c):
        deg_l[pl.ds(i * 16, 16)] = zero16
        return c
    lax.fori_loop(0, ROWS, _zero_deg, 0)

    def _hist16(g, c):
        idx = dvm[pl.ds(g * 16, 16)]
        plsc.addupdate_scatter(deg_l, [idx], ones16)
        return c
    lax.fori_loop(0, EPW // 16, _hist16, 0)

    pltpu.sync_copy(deg_l, out_hbm.at[wid])


@functools.partial(
    pl.kernel,
    out_type=jax.ShapeDtypeStruct((2, N_PAD, 16), jnp.float32),
    mesh=_mesh,
    scratch_types=[
        pltpu.VMEM((NCHUNK, CW), jnp.int32),
        pltpu.VMEM((NCHUNK, CW), jnp.int32),
        pltpu.VMEM((8 * CW, 16), jnp.float32),
        pltpu.VMEM((N_PAD // 16, 16), jnp.float32),
        pltpu.VMEM_SHARED((N_PAD, 16), jnp.float32),
        pltpu.SemaphoreType.DMA((8,)),
        pltpu.SemaphoreType.DMA((8,)),
    ],
    compiler_params=pltpu.CompilerParams(use_tc_tiling_on_sc=False),
)
def _agg_kernel(y_hbm, e4_hbm, out_hbm, svm, dvm, gb, obuf, agg, gsem, ssem):
    cid = lax.axis_index("c")
    sid = lax.axis_index("s")
    wid = cid * 16 + sid
    slab = N_PAD // 16
    pltpu.sync_copy(e4_hbm.at[0, wid], svm)
    pltpu.sync_copy(e4_hbm.at[1, wid], dvm)

    zero16 = jnp.zeros((16,), jnp.float32)

    def _zero_obuf(i, c):
        obuf[i, :] = zero16
        return c
    lax.fori_loop(0, slab, _zero_obuf, 0)
    pltpu.sync_copy(obuf, agg.at[pl.ds(sid * slab, slab)])
    plsc.subcore_barrier()

    NB = 8
    LEAD = 4

    def _group(i, c):
        base = i * NB
        for k in range(NB):
            j = base + k
            buf_k = gb.at[pl.ds(k * CW, CW)]
            km = (k - LEAD) % NB
            buf_m = gb.at[pl.ds(km * CW, CW)]

            @pl.when(j >= NB)
            def _():
                pltpu.make_async_copy(
                    buf_k, agg.at[dvm.at[j - NB]], ssem.at[k]).wait()

            @pl.when(j < NCHUNK)
            def _():
                pltpu.async_copy(y_hbm.at[svm.at[j]], buf_k, gsem.at[k])

            @pl.when(jnp.logical_and(j >= LEAD, j < NCHUNK + LEAD))
            def _():
                pltpu.make_async_copy(
                    y_hbm.at[svm.at[j - LEAD]], buf_m, gsem.at[km]).wait()
                pltpu.async_copy(
                    buf_m, agg.at[dvm.at[j - LEAD]], ssem.at[km], add=True)
        return c
    lax.fori_loop(0, (NCHUNK + NB) // NB, _group, 0)

    plsc.subcore_barrier()
    pltpu.sync_copy(agg.at[pl.ds(sid * slab, slab)], obuf)
    pltpu.sync_copy(obuf, out_hbm.at[cid, pl.ds(sid * slab, slab)])


def _tc1_body(x_ref, w_ref, dall_ref, y_ref, dinv_ref):
    deg = jnp.sum(dall_ref[...], axis=0)[:N, None] + 1.0
    dv = lax.rsqrt(deg)
    xw = jnp.dot(x_ref[...], w_ref[...], preferred_element_type=jnp.float32)
    y_ref[...] = dv * xw
    dinv_ref[...] = dv


_tc1 = pl.pallas_call(
    _tc1_body,
    out_shape=[
        jax.ShapeDtypeStruct((N, 16), jnp.float32),
        jax.ShapeDtypeStruct((N, 1), jnp.float32),
    ],
)


def _tc2_body(a_ref, y1_ref, dinv_ref, b1_ref, w2_ref, y2_ref):
    a = a_ref[...]
    dv = dinv_ref[...]
    pre = (a[0, :N] + a[1, :N] + y1_ref[...]) * dv + b1_ref[...]
    h = jnp.maximum(pre, 0.0)
    hw = jnp.dot(h, w2_ref[...], preferred_element_type=jnp.float32)
    y2_ref[...] = dv * hw


_tc2 = pl.pallas_call(
    _tc2_body,
    out_shape=jax.ShapeDtypeStruct((N, 16), jnp.float32),
)


def _tc3_body(a_ref, y2_ref, dinv_ref, b2_ref, o_ref):
    a = a_ref[...]
    t = (a[0, :N] + a[1, :N] + y2_ref[...]) * dinv_ref[...] + b2_ref[...]
    col = lax.broadcasted_iota(jnp.int32, (N, 16), 1)
    valid = col < D_OUT
    tm = jnp.where(valid, t, jnp.float32(-1e30))
    mx = jnp.max(tm, axis=1, keepdims=True)
    e = jnp.where(valid, jnp.exp(t - mx), 0.0)
    s = jnp.sum(e, axis=1, keepdims=True)
    o_ref[...] = t - mx - jnp.log(s)


_tc3 = pl.pallas_call(
    _tc3_body,
    out_shape=jax.ShapeDtypeStruct((N, 16), jnp.float32),
)


def kernel(x, edge_index, W1, b1, W2, b2):
    e3 = edge_index.reshape(2, NW, EPW)
    e4 = edge_index.reshape(2, NW, NCHUNK, CW)

    w2p = jnp.pad(W2, ((0, 0), (0, 16 - D_OUT)))
    b1p = b1.reshape(1, 16)
    b2p = jnp.pad(b2, (0, 16 - D_OUT)).reshape(1, 16)

    d_all = _deg_kernel(e3)
    y1, dinv = _tc1(x, W1, d_all)
    a = _agg_kernel(y1, e4)
    y2 = _tc2(a, y1, dinv, b1p, w2p)
    a2 = _agg_kernel(y2, e4)
    outf = _tc3(a2, y2, dinv, b2p)
    return outf[:, :D_OUT]


# speedup vs baseline: 1.2982x; 1.1328x over previous
import functools

import jax
import jax.numpy as jnp
from jax import lax
from jax.experimental import pallas as pl
from jax.experimental.pallas import tpu as pltpu
from jax.experimental.pallas import tpu_sc as plsc

N = 10000
E = 320000
D_IN = 128
D_HID = 16
D_OUT = 7

N_PAD = 10240
NW = 32
EPW = E // NW
NCHUNK = 80
CW = EPW // NCHUNK
ROWS = N_PAD // 16

_mesh = plsc.VectorSubcoreMesh(core_axis_name="c", subcore_axis_name="s")


@functools.partial(
    pl.kernel,
    out_type=jax.ShapeDtypeStruct((NW, N_PAD), jnp.float32),
    mesh=_mesh,
    scratch_types=[
        pltpu.VMEM((EPW,), jnp.int32),
        pltpu.VMEM((N_PAD,), jnp.float32),
    ],
    compiler_params=pltpu.CompilerParams(needs_layout_passes=False,
                                         use_tc_tiling_on_sc=False),
)
def _deg_kernel(ei_hbm, out_hbm, dvm, deg_l):
    cid = lax.axis_index("c")
    sid = lax.axis_index("s")
    wid = cid * 16 + sid
    pltpu.sync_copy(ei_hbm.at[1, pl.ds(wid * EPW, EPW)], dvm)

    zero16 = jnp.zeros((16,), jnp.float32)
    ones16 = jnp.ones((16,), jnp.float32)

    def _zero_deg(i, c):
        deg_l[pl.ds(i * 16, 16)] = zero16
        return c
    lax.fori_loop(0, ROWS, _zero_deg, 0)

    def _hist16(g, c):
        idx = dvm[pl.ds(g * 16, 16)]
        plsc.addupdate_scatter(deg_l, [idx], ones16)
        return c
    lax.fori_loop(0, EPW // 16, _hist16, 0)

    pltpu.sync_copy(deg_l, out_hbm.at[wid])


@functools.partial(
    pl.kernel,
    out_type=jax.ShapeDtypeStruct((2, N_PAD, 16), jnp.float32),
    mesh=_mesh,
    scratch_types=[
        pltpu.VMEM((NCHUNK, CW), jnp.int32),
        pltpu.VMEM((NCHUNK, CW), jnp.int32),
        pltpu.VMEM((8 * CW, 16), jnp.float32),
        pltpu.VMEM((N_PAD // 16, 16), jnp.float32),
        pltpu.VMEM_SHARED((N_PAD, 16), jnp.float32),
        pltpu.SemaphoreType.DMA((8,)),
        pltpu.SemaphoreType.DMA((8,)),
    ],
    compiler_params=pltpu.CompilerParams(use_tc_tiling_on_sc=False),
)
def _agg_kernel(y_hbm, e4_hbm, out_hbm, svm, dvm, gb, obuf, agg, gsem, ssem):
    cid = lax.axis_index("c")
    sid = lax.axis_index("s")
    wid = cid * 16 + sid
    slab = N_PAD // 16
    pltpu.sync_copy(e4_hbm.at[0, wid], svm)
    pltpu.sync_copy(e4_hbm.at[1, wid], dvm)

    zero16 = jnp.zeros((16,), jnp.float32)

    def _zero_obuf(i, c):
        obuf[i, :] = zero16
        return c
    lax.fori_loop(0, slab, _zero_obuf, 0)
    pltpu.sync_copy(obuf, agg.at[pl.ds(sid * slab, slab)])
    plsc.subcore_barrier()

    NB = 8
    LEAD = 4

    def _group(i, c):
        base = i * NB
        for k in range(NB):
            j = base + k
            buf_k = gb.at[pl.ds(k * CW, CW)]
            km = (k - LEAD) % NB
            buf_m = gb.at[pl.ds(km * CW, CW)]

            @pl.when(j >= NB)
            def _():
                pltpu.make_async_copy(
                    buf_k, agg.at[dvm.at[j - NB]], ssem.at[k]).wait()

            @pl.when(j < NCHUNK)
            def _():
                pltpu.async_copy(y_hbm.at[svm.at[j]], buf_k, gsem.at[k])

            @pl.when(jnp.logical_and(j >= LEAD, j < NCHUNK + LEAD))
            def _():
                pltpu.make_async_copy(
                    y_hbm.at[svm.at[j - LEAD]], buf_m, gsem.at[km]).wait()
                pltpu.async_copy(
                    buf_m, agg.at[dvm.at[j - LEAD]], ssem.at[km], add=True)
        return c
    lax.fori_loop(0, (NCHUNK + NB) // NB, _group, 0)

    plsc.subcore_barrier()
    pltpu.sync_copy(agg.at[pl.ds(sid * slab, slab)], obuf)
    pltpu.sync_copy(obuf, out_hbm.at[cid, pl.ds(sid * slab, slab)])


NF = N * 16 // 128
AFC = N_PAD * 16 // 128


def _to_flat(v):
    return jnp.concatenate([v[NF * k:NF * (k + 1), :] for k in range(8)],
                           axis=1)


def _from_flat(f):
    return jnp.concatenate([f[:, 16 * k:16 * (k + 1)] for k in range(8)],
                           axis=0)


def _tc1_body(x_ref, w_ref, dall_ref, y_ref, dinv_ref):
    deg = jnp.sum(dall_ref[...], axis=0)[:N, None] + 1.0
    dv = lax.rsqrt(deg)
    xw = jnp.dot(x_ref[...], w_ref[...], preferred_element_type=jnp.float32)
    y_ref[...] = _to_flat(dv * xw)
    dinv_ref[...] = dv


_tc1 = pl.pallas_call(
    _tc1_body,
    out_shape=[
        jax.ShapeDtypeStruct((NF, 128), jnp.float32),
        jax.ShapeDtypeStruct((N, 1), jnp.float32),
    ],
)


def _tc2_body(a_ref, y1_ref, dinv_ref, b1_ref, w2_ref, y2_ref):
    a = _from_flat(a_ref[0, :NF] + a_ref[1, :NF])
    y1 = _from_flat(y1_ref[...])
    dv = dinv_ref[...]
    pre = (a + y1) * dv + b1_ref[...]
    h = jnp.maximum(pre, 0.0)
    hw = jnp.dot(h, w2_ref[...], preferred_element_type=jnp.float32)
    y2_ref[...] = _to_flat(dv * hw)


_tc2 = pl.pallas_call(
    _tc2_body,
    out_shape=jax.ShapeDtypeStruct((NF, 128), jnp.float32),
)


def _tc3_body(a_ref, y2_ref, dinv_ref, b2_ref, o_ref):
    a = _from_flat(a_ref[0, :NF] + a_ref[1, :NF])
    y2 = _from_flat(y2_ref[...])
    t = (a + y2) * dinv_ref[...] + b2_ref[...]
    col = lax.broadcasted_iota(jnp.int32, (N, 16), 1)
    valid = col < D_OUT
    tm = jnp.where(valid, t, jnp.float32(-1e30))
    mx = jnp.max(tm, axis=1, keepdims=True)
    e = jnp.where(valid, jnp.exp(t - mx), 0.0)
    s = jnp.sum(e, axis=1, keepdims=True)
    o_ref[...] = (t - mx - jnp.log(s))[:, :D_OUT]


_tc3 = pl.pallas_call(
    _tc3_body,
    out_shape=jax.ShapeDtypeStruct((N, D_OUT), jnp.float32),
)


def kernel(x, edge_index, W1, b1, W2, b2):
    ep = (edge_index % NF) * 8 + edge_index // NF
    e4p = ep.reshape(2, NW, NCHUNK, CW)

    w2p = jnp.pad(W2, ((0, 0), (0, 16 - D_OUT)))
    b1p = b1.reshape(1, 16)
    b2p = jnp.pad(b2, (0, 16 - D_OUT)).reshape(1, 16)

    d_all = _deg_kernel(edge_index)
    y1f, dinv = _tc1(x, W1, d_all)
    a = _agg_kernel(y1f.reshape(N, 16), e4p)
    y2f = _tc2(a.reshape(2, AFC, 128), y1f, dinv, b1p, w2p)
    a2 = _agg_kernel(y2f.reshape(N, 16), e4p)
    return _tc3(a2.reshape(2, AFC, 128), y2f, dinv, b2p)
